# Initial kernel scaffold; baseline (speedup 1.0000x reference)
import jax
import jax.numpy as jnp
from jax.experimental import pallas as pl


def kernel(scores):
    raise NotImplementedError("write your pallas kernel here")


# speedup vs baseline: 6.3773x; 6.3773x over previous
import functools
import math

import jax
import jax.numpy as jnp
from jax import lax
from jax.experimental import pallas as pl

_K = 512
_ROWS_PER_BLOCK = 8


def _topk_mask_body(x_ref, o_ref):
    x = x_ref[...]
    ui = lax.bitcast_convert_type(x, jnp.uint32)
    ukey = jnp.where(
        ui >= jnp.uint32(0x80000000), ~ui, ui | jnp.uint32(0x80000000)
    )

    def bit_step(i, cand):
        b = 31 - i
        t = cand | (jnp.uint32(1) << b)
        cnt = jnp.sum((ukey >= t).astype(jnp.int32), axis=1, keepdims=True)
        return jnp.where(cnt >= _K, t, cand)

    cand0 = jnp.zeros((x.shape[0], 1), jnp.uint32)
    kth = lax.fori_loop(0, 32, bit_step, cand0)
    o_ref[...] = (ukey >= kth).astype(jnp.float32)


def _topk_mask(s, interpret=False):
    rows, n = s.shape
    grid = (rows // _ROWS_PER_BLOCK,)
    return pl.pallas_call(
        _topk_mask_body,
        grid=grid,
        in_specs=[pl.BlockSpec((_ROWS_PER_BLOCK, n), lambda i: (i, 0))],
        out_specs=pl.BlockSpec((_ROWS_PER_BLOCK, n), lambda i: (i, 0)),
        out_shape=jax.ShapeDtypeStruct((rows, n), jnp.float32),
        interpret=interpret,
    )(s)


@jax.jit
def kernel(scores):
    bsz, nmax, _, ensemble = scores.shape
    n2 = nmax * nmax
    s = jnp.transpose(scores, (0, 3, 1, 2)).reshape(bsz * ensemble, n2)
    hard = _topk_mask(s)
    out = hard.reshape(bsz, ensemble, nmax, nmax)
    return jnp.transpose(out, (0, 2, 3, 1))
